# baseline (speedup 1.0000x reference)
import functools

import jax
import jax.numpy as jnp
from jax import lax
from jax.experimental import pallas as pl
from jax.experimental.pallas import tpu as pltpu
from jax.experimental.pallas import tpu_sc as plsc

VOCAB = 1000000
DIM = 64
NB = 4096
NL = 200

NC = 2
NS = 16
NW = NC * NS
CHUNK = 128
LH = NL // 8
NBUF = 8
DIST = 4
NTBUF = 2


def _make_gather():
  mesh = plsc.VectorSubcoreMesh(core_axis_name="c", subcore_axis_name="s")

  @functools.partial(
      pl.kernel,
      mesh=mesh,
      out_type=jax.ShapeDtypeStruct((NL, 8, NW, 8 * CHUNK), jnp.float32),
      scratch_types=[
          pltpu.VMEM((LH, 8, CHUNK), jnp.int32),
          pltpu.VMEM((NBUF, CHUNK, DIM), jnp.float32),
          pltpu.VMEM((NTBUF, 8, 8 * CHUNK), jnp.float32),
      ] + [pltpu.SemaphoreType.DMA] * (NBUF + NTBUF),
      compiler_params=pltpu.CompilerParams(
          use_tc_tiling_on_sc=False, needs_layout_passes=False),
  )
  def k(idx_hbm, table_hbm, out_hbm, idx_v, emb_v, embt_v, *sems):
    gsem = sems[:NBUF]
    wsem = sems[NBUF:]
    bh = lax.axis_index("s") * NC + lax.axis_index("c")
    pltpu.sync_copy(idx_hbm.at[:, bh], idx_v)

    def gather(lh, ll, b):
      pltpu.async_copy(table_hbm.at[idx_v.at[lh, ll]], emb_v.at[b], gsem[b])

    def wait_gather(b):
      pltpu.make_async_copy(
          table_hbm.at[idx_v.at[0, 0]], emb_v.at[b], gsem[b]).wait()

    def put(l, c):
      pltpu.async_copy(embt_v.at[c], out_hbm.at[l, :, bh], wsem[c])

    def wait_put(c):
      pltpu.make_async_copy(
          embt_v.at[c], out_hbm.at[0, :, bh], wsem[c]).wait()

    toks = [lax.iota(jnp.int32, 16) + tb * 16 for tb in range(8)]

    def transpose(b, c):
      def dh_body(dh, carry):
        f0 = dh * 8
        for dl in range(8):
          feat = jnp.full((16,), f0 + dl, jnp.int32)
          for tb in range(8):
            vals = plsc.load_gather(emb_v.at[b], [toks[tb], feat])
            embt_v[c, dh, pl.ds(dl * CHUNK + tb * 16, 16)] = vals
        return carry

      lax.fori_loop(0, 8, dh_body, 0)

    for ll in range(DIST):
      gather(0, ll, ll)

    def lh_body(lh, carry):
      for ll in range(NBUF):
        l = lh * 8 + ll
        c = ll % NTBUF
        wait_gather(ll)
        if ll < NTBUF:
          @pl.when(lh > 0)
          def _():
            wait_put(c)
        else:
          wait_put(c)
        transpose(ll, c)
        put(l, c)
        nll = (ll + DIST) % NBUF
        if ll < DIST:
          gather(lh, ll + DIST, nll)
        else:

          @pl.when(lh < LH - 1)
          def _():
            gather(lh + 1, nll, nll)

      return carry

    lax.fori_loop(0, LH, lh_body, 0)

    for c in range(NTBUF):
      wait_put(c)

  return k


_gather = _make_gather()


def kernel(input_ids, table):
  idx = (input_ids.astype(jnp.int32)
         .reshape(NW, CHUNK, LH, 8).transpose(2, 0, 3, 1))
  out5 = _gather(idx, table)
  return (out5.reshape(NL, 8, NW, 8, CHUNK)
          .transpose(2, 4, 0, 1, 3).reshape(NB, NL, DIM))


# speedup vs baseline: 1.5413x; 1.5413x over previous
import functools

import jax
import jax.numpy as jnp
from jax import lax
from jax.experimental import pallas as pl
from jax.experimental.pallas import tpu as pltpu
from jax.experimental.pallas import tpu_sc as plsc

VOCAB = 1000000
DIM = 64
NB = 4096
NL = 200

NC = 2
NS = 16
NW = NC * NS
CHUNK = 128
LH = NL // 8
NBUF = 8
DIST = 4


def _make_gather():
  mesh = plsc.VectorSubcoreMesh(core_axis_name="c", subcore_axis_name="s")

  @functools.partial(
      pl.kernel,
      mesh=mesh,
      out_type=jax.ShapeDtypeStruct((NB, NL, DIM), jnp.float32),
      scratch_types=[
          pltpu.VMEM((LH, 8, CHUNK), jnp.int32),
          pltpu.VMEM((NBUF, CHUNK, DIM), jnp.float32),
      ] + [pltpu.SemaphoreType.DMA] * (2 * NBUF),
      compiler_params=pltpu.CompilerParams(
          use_tc_tiling_on_sc=False, needs_layout_passes=False),
  )
  def k(idx_hbm, table_hbm, out_hbm, idx_v, emb_v, *sems):
    gsem = sems[:NBUF]
    wsem = sems[NBUF:]
    bh = lax.axis_index("s") * NC + lax.axis_index("c")
    base = bh * CHUNK
    pltpu.sync_copy(idx_hbm.at[:, bh], idx_v)

    def gather(lh, ll, b):
      pltpu.async_copy(table_hbm.at[idx_v.at[lh, ll]], emb_v.at[b], gsem[b])

    def wait_gather(b):
      pltpu.make_async_copy(
          table_hbm.at[idx_v.at[0, 0]], emb_v.at[b], gsem[b]).wait()

    def put(l, b):
      pltpu.async_copy(
          emb_v.at[b], out_hbm.at[pl.ds(base, CHUNK), l], wsem[b])

    def wait_put(b):
      pltpu.make_async_copy(
          emb_v.at[b], out_hbm.at[pl.ds(base, CHUNK), 0], wsem[b]).wait()

    for ll in range(DIST):
      gather(0, ll, ll)

    def lh_body(lh, carry):
      for ll in range(NBUF):
        l = lh * 8 + ll
        wait_gather(ll)
        put(l, ll)
        nll = (ll + DIST) % NBUF
        if ll < DIST:

          @pl.when(lh > 0)
          def _():
            wait_put(nll)

          gather(lh, ll + DIST, nll)
        else:

          @pl.when(lh < LH - 1)
          def _():
            wait_put(nll)
            gather(lh + 1, nll, nll)

      return carry

    lax.fori_loop(0, LH, lh_body, 0)

    for b in range(NBUF):
      wait_put(b)

  return k


_gather = _make_gather()


def kernel(input_ids, table):
  idx = (input_ids.astype(jnp.int32)
         .reshape(NW, CHUNK, LH, 8).transpose(2, 0, 3, 1))
  return _gather(idx, table)


# speedup vs baseline: 1.6432x; 1.0661x over previous
import functools

import jax
import jax.numpy as jnp
from jax import lax
from jax.experimental import pallas as pl
from jax.experimental.pallas import tpu as pltpu
from jax.experimental.pallas import tpu_sc as plsc

VOCAB = 1000000
DIM = 64
NB = 4096
NL = 200

NC = 2
NS = 16
NW = NC * NS
CHUNK = 128
LH = NL // 8
NBUF = 8
DIST = 4
NTBUF = 2


def _make_gather():
  mesh = plsc.VectorSubcoreMesh(core_axis_name="c", subcore_axis_name="s")

  @functools.partial(
      pl.kernel,
      mesh=mesh,
      out_type=jax.ShapeDtypeStruct((NL, 8, NW, 8 * CHUNK), jnp.float32),
      scratch_types=[
          pltpu.VMEM((LH, 8, CHUNK), jnp.int32),
          pltpu.VMEM((NBUF, CHUNK, DIM), jnp.float32),
          pltpu.VMEM((NTBUF, 8 * CHUNK * 8), jnp.float32),
      ] + [pltpu.SemaphoreType.DMA] * (NBUF + NTBUF),
      compiler_params=pltpu.CompilerParams(
          use_tc_tiling_on_sc=False, needs_layout_passes=False),
  )
  def k(idx_hbm, table_hbm, out_hbm, idx_v, emb_v, embt_v, *sems):
    gsem = sems[:NBUF]
    wsem = sems[NBUF:]
    bh = lax.axis_index("s") * NC + lax.axis_index("c")
    pltpu.sync_copy(idx_hbm.at[:, bh], idx_v)

    def gather(lh, ll, b):
      pltpu.async_copy(table_hbm.at[idx_v.at[lh, ll]], emb_v.at[b], gsem[b])

    def wait_gather(b):
      pltpu.make_async_copy(
          table_hbm.at[idx_v.at[0, 0]], emb_v.at[b], gsem[b]).wait()

    def put(l, c):
      for dh in range(8):
        pltpu.async_copy(embt_v.at[c, pl.ds(dh * 1024, 1024)],
                         out_hbm.at[l, dh, bh], wsem[c])

    def wait_put(c):
      for dh in range(8):
        pltpu.make_async_copy(embt_v.at[c, pl.ds(dh * 1024, 1024)],
                              out_hbm.at[0, dh, bh], wsem[c]).wait()

    iota = lax.iota(jnp.int32, 16)
    rot = [(iota + r) % 16 for r in range(16)]
    toks = [iota + tb * 16 for tb in range(8)]
    srot = [rot[r] * CHUNK + iota for r in range(16)]

    def transpose(b, c):
      def f0_body(q, carry):
        f0 = q * 16
        for tb in range(8):
          for r in range(16):
            vals = plsc.load_gather(emb_v.at[b], [toks[tb], rot[r] + f0])
            plsc.store_scatter(embt_v.at[c],
                               [srot[r] + (f0 * CHUNK + tb * 16)], vals)
        return carry

      lax.fori_loop(0, 4, f0_body, 0)

    for ll in range(DIST):
      gather(0, ll, ll)

    def lh_body(lh, carry):
      for ll in range(NBUF):
        l = lh * 8 + ll
        c = ll % NTBUF
        wait_gather(ll)
        if ll < NTBUF:
          @pl.when(lh > 0)
          def _():
            wait_put(c)
        else:
          wait_put(c)
        transpose(ll, c)
        put(l, c)
        nll = (ll + DIST) % NBUF
        if ll < DIST:
          gather(lh, ll + DIST, nll)
        else:

          @pl.when(lh < LH - 1)
          def _():
            gather(lh + 1, nll, nll)

      return carry

    lax.fori_loop(0, LH, lh_body, 0)

    for c in range(NTBUF):
      wait_put(c)

  return k


_gather = _make_gather()


def kernel(input_ids, table):
  idx = (input_ids.astype(jnp.int32)
         .reshape(NW, CHUNK, LH, 8).transpose(2, 0, 3, 1))
  out5 = _gather(idx, table)
  return (out5.reshape(NL, 8, NW, 8, CHUNK)
          .transpose(2, 4, 0, 1, 3).reshape(NB, NL, DIM))


# speedup vs baseline: 1.9926x; 1.2126x over previous
import functools

import jax
import jax.numpy as jnp
from jax import lax
from jax.experimental import pallas as pl
from jax.experimental.pallas import tpu as pltpu
from jax.experimental.pallas import tpu_sc as plsc

VOCAB = 1000000
DIM = 64
PDIM = 128
NB = 4096
NL = 200

NC = 2
NS = 16
NW = NC * NS
CHUNK = 128
LH = NL // 8
NBUF = 4
DIST = 2


def _make_gather():
  mesh = plsc.VectorSubcoreMesh(core_axis_name="c", subcore_axis_name="s")

  @functools.partial(
      pl.kernel,
      mesh=mesh,
      out_type=jax.ShapeDtypeStruct((NB, NL, PDIM), jnp.float32),
      scratch_types=[
          pltpu.VMEM((LH, 8, CHUNK), jnp.int32),
          pltpu.VMEM((NBUF, CHUNK, PDIM), jnp.float32),
      ] + [pltpu.SemaphoreType.DMA] * (2 * NBUF),
      compiler_params=pltpu.CompilerParams(
          use_tc_tiling_on_sc=False, needs_layout_passes=False),
  )
  def k(idx_hbm, table_hbm, out_hbm, idx_v, emb_v, *sems):
    gsem = sems[:NBUF]
    wsem = sems[NBUF:]
    bh = lax.axis_index("s") * NC + lax.axis_index("c")
    base = bh * CHUNK
    pltpu.sync_copy(idx_hbm.at[:, bh], idx_v)

    def gather(lh, ll, b):
      pltpu.async_copy(table_hbm.at[idx_v.at[lh, ll]], emb_v.at[b], gsem[b])

    def wait_gather(b):
      pltpu.make_async_copy(
          table_hbm.at[idx_v.at[0, 0]], emb_v.at[b], gsem[b]).wait()

    def put(l, b):
      pltpu.async_copy(
          emb_v.at[b, :, pl.ds(0, DIM)],
          out_hbm.at[pl.ds(base, CHUNK), l, pl.ds(0, DIM)], wsem[b])

    def wait_put(b):
      pltpu.make_async_copy(
          emb_v.at[b, :, pl.ds(0, DIM)],
          out_hbm.at[pl.ds(base, CHUNK), 0, pl.ds(0, DIM)], wsem[b]).wait()

    for ll in range(DIST):
      gather(0, ll, ll)

    def lh_body(lh, carry):
      for ll in range(8):
        lb = ll % NBUF
        l = lh * 8 + ll
        wait_gather(lb)
        put(l, lb)
        nlb = (lb + DIST) % NBUF
        if ll < 8 - DIST:

          @pl.when(jnp.logical_or(lh > 0, ll >= DIST))
          def _():
            wait_put(nlb)

          gather(lh, ll + DIST, nlb)
        else:

          @pl.when(lh < LH - 1)
          def _():
            wait_put(nlb)
            gather(lh + 1, ll + DIST - 8, nlb)

      return carry

    lax.fori_loop(0, LH, lh_body, 0)

    for b in range(NBUF):
      wait_put(b)

  return k


_gather = _make_gather()


def kernel(input_ids, table):
  idx = (input_ids.astype(jnp.int32)
         .reshape(NW, CHUNK, LH, 8).transpose(2, 0, 3, 1))
  tableP = jnp.pad(table, ((0, 0), (0, PDIM - DIM)))
  outP = _gather(idx, tableP)
  return outP[:, :, :DIM]


# speedup vs baseline: 2.2042x; 1.1062x over previous
import functools

import jax
import jax.numpy as jnp
from jax import lax
from jax.experimental import pallas as pl
from jax.experimental.pallas import tpu as pltpu
from jax.experimental.pallas import tpu_sc as plsc

VOCAB = 1000000
DIM = 64
PDIM = 128
NB = 4096
NL = 200

NC = 2
NS = 16
NW = NC * NS
CHUNK = 128
LH = NL // 8
NBUF = 8
DIST = 4


def _make_gather():
  mesh = plsc.VectorSubcoreMesh(core_axis_name="c", subcore_axis_name="s")

  @functools.partial(
      pl.kernel,
      mesh=mesh,
      out_type=jax.ShapeDtypeStruct((NB, NL, PDIM), jnp.float32),
      scratch_types=[
          pltpu.VMEM((LH, 8, CHUNK), jnp.int32),
          pltpu.VMEM((NBUF, CHUNK, DIM), jnp.float32),
      ] + [pltpu.SemaphoreType.DMA] * (2 * NBUF),
      compiler_params=pltpu.CompilerParams(
          use_tc_tiling_on_sc=False, needs_layout_passes=False),
  )
  def k(idx_hbm, table_hbm, out_hbm, idx_v, emb_v, *sems):
    gsem = sems[:NBUF]
    wsem = sems[NBUF:]
    bh = lax.axis_index("s") * NC + lax.axis_index("c")
    base = bh * CHUNK
    pltpu.sync_copy(idx_hbm.at[:, bh], idx_v)

    def dbl_body(lh2, carry):
      for ll in range(8):
        for kk in range(8):
          s = idx_v[lh2, ll, pl.ds(kk * 16, 16)]
          idx_v[lh2, ll, pl.ds(kk * 16, 16)] = s + s
      return carry

    lax.fori_loop(0, LH, dbl_body, 0)

    def gather(lh, ll, b):
      pltpu.async_copy(table_hbm.at[idx_v.at[lh, ll]], emb_v.at[b], gsem[b])

    def wait_gather(b):
      pltpu.make_async_copy(
          table_hbm.at[idx_v.at[0, 0]], emb_v.at[b], gsem[b]).wait()

    def put(l, b):
      pltpu.async_copy(
          emb_v.at[b],
          out_hbm.at[pl.ds(base, CHUNK), l, pl.ds(0, DIM)], wsem[b])

    def wait_put(b):
      pltpu.make_async_copy(
          emb_v.at[b],
          out_hbm.at[pl.ds(base, CHUNK), 0, pl.ds(0, DIM)], wsem[b]).wait()

    for ll in range(DIST):
      gather(0, ll, ll)

    def lh_body(lh, carry):
      for ll in range(8):
        lb = ll % NBUF
        l = lh * 8 + ll
        wait_gather(lb)
        put(l, lb)
        nlb = (lb + DIST) % NBUF
        if ll < 8 - DIST:

          @pl.when(lh > 0)
          def _():
            wait_put(nlb)

          gather(lh, ll + DIST, nlb)
        else:

          @pl.when(lh < LH - 1)
          def _():
            wait_put(nlb)
            gather(lh + 1, ll + DIST - 8, nlb)

      return carry

    lax.fori_loop(0, LH, lh_body, 0)

    for b in range(NBUF):
      wait_put(b)

  return k


_gather = _make_gather()


def kernel(input_ids, table):
  idx = (input_ids.astype(jnp.int32)
         .reshape(NW, CHUNK, LH, 8).transpose(2, 0, 3, 1))
  tableP = jnp.pad(table, ((0, 0), (0, PDIM - DIM))).reshape(2 * VOCAB, DIM)
  outP = _gather(idx, tableP)
  return outP[:, :, :DIM]
